# baseline (speedup 1.0000x reference)
import functools

import jax
import jax.numpy as jnp
from jax import lax
from jax.experimental import pallas as pl
from jax.experimental.pallas import tpu as pltpu
from jax.experimental.pallas import tpu_sc as plsc

N = 10000
E = 320000
H = 128
LAYERS = 4
C = 7
CP = 8

NC = 2
NS = 16
NW = NC * NS

CHUNK = 128
NBUF = 2
CPW0 = 156
CPW1 = 2
E0 = NS * CPW0 * CHUNK
E0_REAL = E - NS * CPW1 * CHUNK
GROUPS0 = CPW0 // NBUF
GROUPS1 = CPW1 // NBUF

ACC_ROWS = 10240
RPS = ACC_ROWS // NS
WB = RPS // CHUNK


def _sc_segment_sum(h, src3, dst3):
    mesh = plsc.VectorSubcoreMesh(core_axis_name="c", subcore_axis_name="s")

    @functools.partial(
        pl.kernel,
        mesh=mesh,
        out_type=jax.ShapeDtypeStruct((NC, ACC_ROWS, H), jnp.float32),
        scratch_types=[
            pltpu.VMEM_SHARED((ACC_ROWS, H), jnp.float32),
            pltpu.VMEM((CHUNK, H), jnp.float32),
            pltpu.VMEM((CHUNK, H), jnp.float32),
            pltpu.VMEM((CHUNK,), jnp.int32),
            pltpu.VMEM((CHUNK,), jnp.int32),
            pltpu.VMEM((CHUNK,), jnp.int32),
            pltpu.VMEM((CHUNK,), jnp.int32),
            pltpu.SemaphoreType.DMA,
            pltpu.SemaphoreType.DMA,
            pltpu.SemaphoreType.DMA,
            pltpu.SemaphoreType.DMA,
            pltpu.SemaphoreType.DMA,
            pltpu.SemaphoreType.DMA,
        ],
    )
    def sc_kernel(h_hbm, src_hbm, dst_hbm, out_hbm, acc,
                  r0, r1, is0, is1, id0, id1,
                  sg0, sg1, si0, si1, sj0, sj1):
        c = lax.axis_index("c")
        s = lax.axis_index("s")
        wid = c * NS + s
        cpw = jnp.where(c == 0, CPW0, CPW1)
        groups = jnp.where(c == 0, GROUPS0, GROUPS1)
        rbufs = (r0, r1)
        isb = (is0, is1)
        idb = (id0, id1)
        sg = (sg0, sg1)
        si = (si0, si1)
        sj = (sj0, sj1)

        zeros16 = jnp.zeros((16,), jnp.float32)

        def zero_row(r, carry):
            for k in range(H // 16):
                r0[r, pl.ds(k * 16, 16)] = zeros16
            return carry

        lax.fori_loop(0, CHUNK, zero_row, 0)
        base = s * RPS
        for i in range(WB):
            pltpu.sync_copy(r0, acc.at[pl.ds(base + i * CHUNK, CHUNK)])
        plsc.subcore_barrier()

        for b in range(NBUF):
            pltpu.async_copy(src_hbm.at[wid, b], isb[b], si[b])
            pltpu.async_copy(dst_hbm.at[wid, b], idb[b], sj[b])
        pltpu.make_async_copy(src_hbm.at[wid, 0], isb[0], si[0]).wait()
        pltpu.async_copy(h_hbm.at[isb[0]], rbufs[0], sg[0])

        def group(g, carry):
            for b in range(NBUF):
                ci = g * NBUF + b
                bn = (b + 1) % NBUF

                @pl.when(ci + 1 < cpw)
                def _():
                    pltpu.make_async_copy(
                        src_hbm.at[wid, ci + 1], isb[bn], si[bn]).wait()
                    pltpu.async_copy(h_hbm.at[isb[bn]], rbufs[bn], sg[bn])

                pltpu.make_async_copy(
                    h_hbm.at[isb[b]], rbufs[b], sg[b]).wait()
                pltpu.make_async_copy(
                    dst_hbm.at[wid, ci], idb[b], sj[b]).wait()
                pltpu.sync_copy(rbufs[b], acc.at[idb[b]], add=True)

                @pl.when(ci + NBUF < cpw)
                def _():
                    pltpu.async_copy(src_hbm.at[wid, ci + NBUF],
                                     isb[b], si[b])
                    pltpu.async_copy(dst_hbm.at[wid, ci + NBUF],
                                     idb[b], sj[b])
            return carry

        lax.fori_loop(0, groups, group, 0)
        plsc.subcore_barrier()
        for i in range(WB):
            off = base + i * CHUNK
            pltpu.sync_copy(acc.at[pl.ds(off, CHUNK)],
                            out_hbm.at[c, pl.ds(off, CHUNK)])

    return sc_kernel(h, src3, dst3)


def _bn(x, g, b):
    m = jnp.mean(x, axis=0)
    v = jnp.mean((x - m) ** 2, axis=0)
    return (x - m) / jnp.sqrt(v + 1e-5) * g + b


def _dense_body(h_ref, acc_ref, w1_ref, b1_ref, g1_ref, bb1_ref, w2_ref,
                b2_ref, ag_ref, ab_ref, ng_ref, nb_ref, pw_ref, pb_ref,
                score_ref, hout_ref, sout_ref):
    h = h_ref[...]
    z = h + acc_ref[0, :N, :] + acc_ref[1, :N, :]
    t = jnp.dot(z, w1_ref[...], preferred_element_type=jnp.float32)
    t = jnp.maximum(_bn(t + b1_ref[...], g1_ref[...], bb1_ref[...]), 0.0)
    u = jnp.dot(t, w2_ref[...], preferred_element_type=jnp.float32)
    u = jnp.maximum(_bn(u + b2_ref[...], ag_ref[...], ab_ref[...]), 0.0)
    u = jnp.maximum(_bn(u, ng_ref[...], nb_ref[...]), 0.0)
    hn = h + u
    hout_ref[...] = hn
    sout_ref[...] = (score_ref[...]
                     + jnp.dot(hn, pw_ref[...],
                               preferred_element_type=jnp.float32)
                     + pb_ref[...])


def _dense_layer(h, acc, w1, b1, g1, bb1, w2, b2, ag, ab, ng, nb, pw, pb,
                 score):
    return pl.pallas_call(
        _dense_body,
        out_shape=(
            jax.ShapeDtypeStruct((N, H), jnp.float32),
            jax.ShapeDtypeStruct((N, CP), jnp.float32),
        ),
    )(h, acc, w1, b1, g1, bb1, w2, b2, ag, ab, ng, nb, pw, pb, score)


def _embed_body(h_ref, w_ref, b_ref, pw_ref, pb_ref, hout_ref, sout_ref):
    h = jnp.dot(h_ref[...], w_ref[...],
                preferred_element_type=jnp.float32) + b_ref[...]
    hout_ref[...] = h
    sout_ref[...] = jnp.dot(h, pw_ref[...],
                            preferred_element_type=jnp.float32) + pb_ref[...]


def _embed(h0, w, b, pw, pb):
    return pl.pallas_call(
        _embed_body,
        out_shape=(
            jax.ShapeDtypeStruct((N, H), jnp.float32),
            jax.ShapeDtypeStruct((N, CP), jnp.float32),
        ),
    )(h0, w, b, pw, pb)


def kernel(h, edge_index, e, emb_W, emb_b, mlp_W1, mlp_b1, mlp_g1, mlp_bb1,
           mlp_W2, mlp_b2, apply_g, apply_b, norm_g, norm_b, pred_W, pred_b):
    del e
    src = edge_index[0]
    dst = edge_index[1]
    pad0 = E0 - E0_REAL
    src0 = jnp.concatenate(
        [src[:E0_REAL], jnp.zeros((pad0,), jnp.int32)]).reshape(
            NS, CPW0, CHUNK)
    dst0 = jnp.concatenate(
        [dst[:E0_REAL], jnp.full((pad0,), N, jnp.int32)]).reshape(
            NS, CPW0, CHUNK)
    src1 = src[E0_REAL:].reshape(NS, CPW1, CHUNK)
    dst1 = dst[E0_REAL:].reshape(NS, CPW1, CHUNK)
    zpad = ((0, 0), (0, CPW0 - CPW1), (0, 0))
    src3 = jnp.concatenate(
        [src0, jnp.pad(src1, zpad)], axis=0)
    dst3 = jnp.concatenate(
        [dst0, jnp.pad(dst1, zpad, constant_values=N)], axis=0)

    pwp = jnp.pad(pred_W, ((0, 0), (0, 0), (0, CP - C)))
    pbp = jnp.pad(pred_b, ((0, 0), (0, CP - C)))

    hcur, score = _embed(h, emb_W, emb_b, pwp[0], pbp[0])
    for i in range(LAYERS):
        acc = _sc_segment_sum(hcur, src3, dst3)
        hcur, score = _dense_layer(
            hcur, acc, mlp_W1[i], mlp_b1[i], mlp_g1[i], mlp_bb1[i],
            mlp_W2[i], mlp_b2[i], apply_g[i], apply_b[i], norm_g[i],
            norm_b[i], pwp[i + 1], pbp[i + 1], score)
    return score[:, :C]


# speedup vs baseline: 1.4123x; 1.4123x over previous
import functools

import jax
import jax.numpy as jnp
from jax import lax
from jax.experimental import pallas as pl
from jax.experimental.pallas import tpu as pltpu
from jax.experimental.pallas import tpu_sc as plsc

N = 10000
E = 320000
H = 128
LAYERS = 4
C = 7
CP = 8

NC = 2
NS = 16
NW = NC * NS

CHUNK = 80
NBUF = 3
CPW0 = 168
CPW1 = 84
E0 = NS * CPW0 * CHUNK
E0_REAL = E - NS * CPW1 * CHUNK
GROUPS0 = CPW0 // NBUF
GROUPS1 = CPW1 // NBUF

ACC_ROWS = 10240
RPS = ACC_ROWS // NS
WB = RPS // CHUNK


def _sc_segment_sum(h, src3, dst3):
    mesh = plsc.VectorSubcoreMesh(core_axis_name="c", subcore_axis_name="s")

    @functools.partial(
        pl.kernel,
        mesh=mesh,
        out_type=jax.ShapeDtypeStruct((NC, ACC_ROWS, H), jnp.float32),
        scratch_types=[
            pltpu.VMEM_SHARED((ACC_ROWS, H), jnp.float32),
            pltpu.VMEM((CHUNK, H), jnp.float32),
            pltpu.VMEM((CHUNK, H), jnp.float32),
            pltpu.VMEM((CHUNK, H), jnp.float32),
            pltpu.VMEM((CHUNK,), jnp.int32),
            pltpu.VMEM((CHUNK,), jnp.int32),
            pltpu.VMEM((CHUNK,), jnp.int32),
            pltpu.VMEM((CHUNK,), jnp.int32),
            pltpu.VMEM((CHUNK,), jnp.int32),
            pltpu.VMEM((CHUNK,), jnp.int32),
            pltpu.SemaphoreType.DMA,
            pltpu.SemaphoreType.DMA,
            pltpu.SemaphoreType.DMA,
            pltpu.SemaphoreType.DMA,
            pltpu.SemaphoreType.DMA,
            pltpu.SemaphoreType.DMA,
            pltpu.SemaphoreType.DMA,
            pltpu.SemaphoreType.DMA,
            pltpu.SemaphoreType.DMA,
        ],
    )
    def sc_kernel(h_hbm, src_hbm, dst_hbm, out_hbm, acc,
                  r0, r1, r2, is0, is1, is2, id0, id1, id2,
                  sg0, sg1, sg2, si0, si1, si2, sj0, sj1, sj2):
        c = lax.axis_index("c")
        s = lax.axis_index("s")
        wid = c * NS + s
        cpw = jnp.where(c == 0, CPW0, CPW1)
        groups = jnp.where(c == 0, GROUPS0, GROUPS1)
        rbufs = (r0, r1, r2)
        isb = (is0, is1, is2)
        idb = (id0, id1, id2)
        sg = (sg0, sg1, sg2)
        si = (si0, si1, si2)
        sj = (sj0, sj1, sj2)

        zeros16 = jnp.zeros((16,), jnp.float32)

        def zero_row(r, carry):
            for k in range(H // 16):
                r0[r, pl.ds(k * 16, 16)] = zeros16
            return carry

        lax.fori_loop(0, CHUNK, zero_row, 0)
        base = s * RPS
        for i in range(WB):
            pltpu.sync_copy(r0, acc.at[pl.ds(base + i * CHUNK, CHUNK)])
        plsc.subcore_barrier()

        for b in range(NBUF):
            pltpu.async_copy(src_hbm.at[wid, b], isb[b], si[b])
            pltpu.async_copy(dst_hbm.at[wid, b], idb[b], sj[b])
        pltpu.make_async_copy(src_hbm.at[wid, 0], isb[0], si[0]).wait()
        pltpu.async_copy(h_hbm.at[isb[0]], rbufs[0], sg[0])

        def group(g, carry):
            for b in range(NBUF):
                ci = g * NBUF + b
                bn = (b + 1) % NBUF

                @pl.when(ci + 1 < cpw)
                def _():
                    pltpu.make_async_copy(
                        src_hbm.at[wid, ci + 1], isb[bn], si[bn]).wait()
                    pltpu.async_copy(h_hbm.at[isb[bn]], rbufs[bn], sg[bn])

                pltpu.make_async_copy(
                    h_hbm.at[isb[b]], rbufs[b], sg[b]).wait()
                pltpu.make_async_copy(
                    dst_hbm.at[wid, ci], idb[b], sj[b]).wait()
                pltpu.sync_copy(rbufs[b], acc.at[idb[b]], add=True)

                @pl.when(ci + NBUF < cpw)
                def _():
                    pltpu.async_copy(src_hbm.at[wid, ci + NBUF],
                                     isb[b], si[b])
                    pltpu.async_copy(dst_hbm.at[wid, ci + NBUF],
                                     idb[b], sj[b])
            return carry

        lax.fori_loop(0, groups, group, 0)
        plsc.subcore_barrier()
        for i in range(WB):
            off = base + i * CHUNK
            pltpu.sync_copy(acc.at[pl.ds(off, CHUNK)],
                            out_hbm.at[c, pl.ds(off, CHUNK)])

    return sc_kernel(h, src3, dst3)


def _bn(x, g, b):
    m = jnp.mean(x, axis=0)
    v = jnp.mean((x - m) ** 2, axis=0)
    return (x - m) / jnp.sqrt(v + 1e-5) * g + b


def _dense_body(h_ref, acc_ref, w1_ref, b1_ref, g1_ref, bb1_ref, w2_ref,
                b2_ref, ag_ref, ab_ref, ng_ref, nb_ref, pw_ref, pb_ref,
                score_ref, hout_ref, sout_ref):
    h = h_ref[...]
    z = h + acc_ref[0, :N, :] + acc_ref[1, :N, :]
    t = jnp.dot(z, w1_ref[...], preferred_element_type=jnp.float32)
    t = jnp.maximum(_bn(t + b1_ref[...], g1_ref[...], bb1_ref[...]), 0.0)
    u = jnp.dot(t, w2_ref[...], preferred_element_type=jnp.float32)
    u = jnp.maximum(_bn(u + b2_ref[...], ag_ref[...], ab_ref[...]), 0.0)
    u = jnp.maximum(_bn(u, ng_ref[...], nb_ref[...]), 0.0)
    hn = h + u
    hout_ref[...] = hn
    sout_ref[...] = (score_ref[...]
                     + jnp.dot(hn, pw_ref[...],
                               preferred_element_type=jnp.float32)
                     + pb_ref[...])


def _dense_layer(h, acc, w1, b1, g1, bb1, w2, b2, ag, ab, ng, nb, pw, pb,
                 score):
    return pl.pallas_call(
        _dense_body,
        out_shape=(
            jax.ShapeDtypeStruct((N, H), jnp.float32),
            jax.ShapeDtypeStruct((N, CP), jnp.float32),
        ),
    )(h, acc, w1, b1, g1, bb1, w2, b2, ag, ab, ng, nb, pw, pb, score)


def _embed_body(h_ref, w_ref, b_ref, pw_ref, pb_ref, hout_ref, sout_ref):
    h = jnp.dot(h_ref[...], w_ref[...],
                preferred_element_type=jnp.float32) + b_ref[...]
    hout_ref[...] = h
    sout_ref[...] = jnp.dot(h, pw_ref[...],
                            preferred_element_type=jnp.float32) + pb_ref[...]


def _embed(h0, w, b, pw, pb):
    return pl.pallas_call(
        _embed_body,
        out_shape=(
            jax.ShapeDtypeStruct((N, H), jnp.float32),
            jax.ShapeDtypeStruct((N, CP), jnp.float32),
        ),
    )(h0, w, b, pw, pb)


def kernel(h, edge_index, e, emb_W, emb_b, mlp_W1, mlp_b1, mlp_g1, mlp_bb1,
           mlp_W2, mlp_b2, apply_g, apply_b, norm_g, norm_b, pred_W, pred_b):
    del e
    src = edge_index[0]
    dst = edge_index[1]
    pad0 = E0 - E0_REAL
    src0 = jnp.concatenate(
        [src[:E0_REAL], jnp.zeros((pad0,), jnp.int32)]).reshape(
            NS, CPW0, CHUNK)
    dst0 = jnp.concatenate(
        [dst[:E0_REAL], jnp.full((pad0,), N, jnp.int32)]).reshape(
            NS, CPW0, CHUNK)
    src1 = src[E0_REAL:].reshape(NS, CPW1, CHUNK)
    dst1 = dst[E0_REAL:].reshape(NS, CPW1, CHUNK)
    zpad = ((0, 0), (0, CPW0 - CPW1), (0, 0))
    src3 = jnp.concatenate(
        [src0, jnp.pad(src1, zpad)], axis=0)
    dst3 = jnp.concatenate(
        [dst0, jnp.pad(dst1, zpad, constant_values=N)], axis=0)

    pwp = jnp.pad(pred_W, ((0, 0), (0, 0), (0, CP - C)))
    pbp = jnp.pad(pred_b, ((0, 0), (0, CP - C)))

    hcur, score = _embed(h, emb_W, emb_b, pwp[0], pbp[0])
    for i in range(LAYERS):
        acc = _sc_segment_sum(hcur, src3, dst3)
        hcur, score = _dense_layer(
            hcur, acc, mlp_W1[i], mlp_b1[i], mlp_g1[i], mlp_bb1[i],
            mlp_W2[i], mlp_b2[i], apply_g[i], apply_b[i], norm_g[i],
            norm_b[i], pwp[i + 1], pbp[i + 1], score)
    return score[:, :C]
